# baseline (speedup 1.0000x reference)
import functools

import jax
import jax.numpy as jnp
from jax import lax
from jax.experimental import pallas as pl
from jax.experimental.pallas import tpu as pltpu
from jax.experimental.pallas import tpu_sc as plsc


def _make_sc_kernel(B, T, D):
    info = plsc.get_sparse_core_info()
    NC, NS = info.num_cores, info.num_subcores
    NW = NC * NS
    total = B * T * D
    per_w = total // NW
    CH = 49152
    n_chunks = per_w // CH
    UNROLL = 16
    mesh = plsc.VectorSubcoreMesh(core_axis_name="c", subcore_axis_name="s")

    @functools.partial(
        pl.kernel,
        mesh=mesh,
        out_type=jax.ShapeDtypeStruct((total,), jnp.float32),
        scratch_types=[
            pltpu.VMEM((CH,), jnp.float32),
            pltpu.VMEM((CH,), jnp.float32),
        ],
    )
    def k(x_hbm, r_hbm, out_hbm, xbuf, rbuf):
        wid = lax.axis_index("s") * NC + lax.axis_index("c")
        base = wid * per_w
        rbase = base % (T * D)
        for c in range(n_chunks):
            off = base + c * CH
            roff = rbase + c * CH
            pltpu.sync_copy(x_hbm.at[pl.ds(off, CH)], xbuf)
            pltpu.sync_copy(r_hbm.at[pl.ds(roff, CH)], rbuf)

            def body(n, _):
                s = n * (16 * UNROLL)
                for u in range(UNROLL):
                    sl = pl.ds(s + u * 16, 16)
                    xbuf[sl] = xbuf[sl] + rbuf[sl]
                return 0

            lax.fori_loop(0, CH // (16 * UNROLL), body, 0)
            pltpu.sync_copy(xbuf, out_hbm.at[pl.ds(off, CH)])

    return k


def kernel(x, rank_emb):
    B, T, D = x.shape
    k = _make_sc_kernel(B, T, D)
    out = k(x.reshape(-1), rank_emb[:T].reshape(-1))
    return out.reshape(B, T, D)


# speedup vs baseline: 5.9161x; 5.9161x over previous
import jax
import jax.numpy as jnp
from jax.experimental import pallas as pl
from jax.experimental.pallas import tpu as pltpu


_TB = 512


def _add_kernel(x_ref, r_ref, o_ref):
    o_ref[...] = x_ref[...] + r_ref[None]


def kernel(x, rank_emb):
    B, T, D = x.shape
    tb = _TB if T % _TB == 0 else T
    bb = B if B <= 4 else 1
    grid = (T // tb, B // bb)
    return pl.pallas_call(
        _add_kernel,
        grid=grid,
        in_specs=[
            pl.BlockSpec((bb, tb, D), lambda t, b: (b, t, 0)),
            pl.BlockSpec((tb, D), lambda t, b: (t, 0)),
        ],
        out_specs=pl.BlockSpec((bb, tb, D), lambda t, b: (b, t, 0)),
        out_shape=jax.ShapeDtypeStruct((B, T, D), x.dtype),
        compiler_params=pltpu.CompilerParams(
            dimension_semantics=("parallel", "parallel"),
        ),
    )(x, rank_emb[:T])


# speedup vs baseline: 5.9630x; 1.0079x over previous
import jax
import jax.numpy as jnp
from jax.experimental import pallas as pl
from jax.experimental.pallas import tpu as pltpu


_TB = 1024


def _add_kernel(x_ref, r_ref, o_ref):
    o_ref[...] = x_ref[...] + r_ref[None]


def kernel(x, rank_emb):
    B, T, D = x.shape
    tb = _TB if T % _TB == 0 else T
    bb = B if B <= 4 else 1
    grid = (T // tb, B // bb)
    return pl.pallas_call(
        _add_kernel,
        grid=grid,
        in_specs=[
            pl.BlockSpec((bb, tb, D), lambda t, b: (b, t, 0)),
            pl.BlockSpec((tb, D), lambda t, b: (t, 0)),
        ],
        out_specs=pl.BlockSpec((bb, tb, D), lambda t, b: (b, t, 0)),
        out_shape=jax.ShapeDtypeStruct((B, T, D), x.dtype),
        compiler_params=pltpu.CompilerParams(
            dimension_semantics=("parallel", "parallel"),
        ),
    )(x, rank_emb[:T])
